# baseline (speedup 1.0000x reference)
import functools

import jax
import jax.numpy as jnp
from jax import lax
from jax.experimental import pallas as pl
from jax.experimental.pallas import tpu as pltpu
from jax.experimental.pallas import tpu_sc as plsc

N = 4096
D = 1024
E = 8
FF = 2048
K = 2
NA = N * K
BLK = 256
G = (NA + E * (BLK - 1) + BLK - 1) // BLK
GPAD = 128
GROWS = G * BLK

_NEG = -1e30


def _router_math(x2d, wg):
    scores = lax.dot_general(x2d, wg, (((1,), (1,)), ((), ())),
                             preferred_element_type=jnp.float32)
    iota_e = lax.broadcasted_iota(jnp.int32, (N, E), 1)
    m0 = jnp.max(scores, axis=1, keepdims=True)
    e0 = jnp.min(jnp.where(scores >= m0, iota_e, E), axis=1, keepdims=True)
    sc1 = jnp.where(iota_e == e0, _NEG, scores)
    m1 = jnp.max(sc1, axis=1, keepdims=True)
    e1 = jnp.min(jnp.where(sc1 >= m1, iota_e, E), axis=1, keepdims=True)
    z = jnp.exp(m1 - m0)
    p0 = 1.0 / (1.0 + z)
    p1 = z / (1.0 + z)

    oh0 = (iota_e == e0).astype(jnp.float32)
    oh1 = (iota_e == e1).astype(jnp.float32)
    a = jnp.concatenate([oh0, oh1], axis=0)
    incl = a
    s = 1
    while s < NA:
        incl = incl + jnp.concatenate(
            [jnp.zeros((s, E), jnp.float32), incl[:NA - s]], axis=0)
        s *= 2
    rank = jnp.sum(a * incl, axis=1, keepdims=True) - 1.0
    counts = incl[NA - 1:NA, :]
    ci = counts.astype(jnp.int32)
    pc = ((ci + (BLK - 1)) // BLK) * BLK
    pcf = pc.astype(jnp.float32)
    ex = jnp.concatenate([jnp.zeros((1, 1), jnp.float32), pcf[:, :E - 1]],
                         axis=1)
    s = 1
    while s < E:
        ex = ex + jnp.concatenate(
            [jnp.zeros((1, s), jnp.float32), ex[:, :E - s]], axis=1)
        s *= 2
    pos_f = jnp.sum(a * ex, axis=1, keepdims=True) + rank
    pos = pos_f.astype(jnp.int32)
    gstart = lax.broadcasted_iota(jnp.int32, (1, GPAD), 1).astype(
        jnp.float32) * float(BLK)
    be = jnp.zeros((1, GPAD), jnp.float32)
    for e in range(E):
        be = be + (gstart >= ex[:, e:e + 1]).astype(jnp.float32)
    be_i = be.astype(jnp.int32) - 1
    nonempty = pc > 0
    lne = jnp.zeros((1, 1), jnp.int32)
    for e in range(E):
        lne = jnp.where(nonempty[:, e:e + 1], e, lne)
    be_i = jnp.minimum(be_i, lne)
    nne_rows = []
    for e in range(E):
        nxt_e = jnp.full((1, 1), E, jnp.int32)
        for ep in range(E - 1, e, -1):
            nxt_e = jnp.where(nonempty[:, ep:ep + 1], ep, nxt_e)
        nne_rows.append(nxt_e)
    nne = jnp.concatenate(nne_rows, axis=0)
    cmp = lax.broadcasted_iota(jnp.int32, (E, GPAD), 0) == be_i
    nxte = jnp.sum(jnp.where(cmp, nne, 0), axis=0, keepdims=True)
    chg = jnp.concatenate(
        [jnp.ones((1, 1), jnp.float32),
         (be_i[:, 1:] != be_i[:, :-1]).astype(jnp.float32)], axis=1)
    csum = chg
    s = 1
    while s < GPAD:
        csum = csum + jnp.concatenate(
            [jnp.zeros((1, s), jnp.float32), csum[:, :GPAD - s]], axis=1)
        s *= 2
    slot = (csum.astype(jnp.int32) - 1) % 2
    nblk = jnp.sum(pc, axis=1, keepdims=True) // BLK
    nblk = jnp.broadcast_to(nblk, (1, GPAD))
    pflat = jnp.concatenate([p0, p1], axis=0)
    return pos, pflat, be_i, slot, nxte, nblk


def _router_body(x_ref, wg_ref, pos_ref, pf_ref, be_ref, sl_ref, nx_ref,
                 nb_ref):
    pos, pflat, be, slot, nxte, nblk = _router_math(x_ref[...], wg_ref[...])
    pos_ref[...] = pos.reshape(NA // 128, 128)
    pf_ref[...] = pflat.reshape(NA // 128, 128)
    be_ref[...] = be
    sl_ref[...] = slot
    nx_ref[...] = nxte
    nb_ref[...] = nblk


_router_call = pl.pallas_call(
    _router_body,
    out_shape=[
        jax.ShapeDtypeStruct((NA // 128, 128), jnp.int32),
        jax.ShapeDtypeStruct((NA // 128, 128), jnp.float32),
        jax.ShapeDtypeStruct((1, GPAD), jnp.int32),
        jax.ShapeDtypeStruct((1, GPAD), jnp.int32),
        jax.ShapeDtypeStruct((1, GPAD), jnp.int32),
        jax.ShapeDtypeStruct((1, GPAD), jnp.int32),
    ],
)


_NC = 2
_NS = 16
_NW = _NC * _NS
_APW = NA // _NW
_DCH = 32
_CCH = 16

@functools.cache
def _sc_mesh():
    return plsc.VectorSubcoreMesh(core_axis_name="c", subcore_axis_name="s",
                                  num_cores=_NC, num_subcores=_NS)


@functools.cache
def _dispatch_call():
    return functools.partial(
        pl.kernel,
        mesh=_sc_mesh(),
        out_type=jax.ShapeDtypeStruct((GROWS, D), jnp.float32),
        scratch_types=[
            pltpu.VMEM((_APW // _DCH, _DCH), jnp.int32),
            pltpu.VMEM((2, _DCH, D), jnp.float32),
            pltpu.SemaphoreType.DMA((2,)),
        ],
    )(_dispatch_body)


def _dispatch_body(x_hbm, pos_hbm, xs_hbm, idx_v, rows_v, sems):
    wid = lax.axis_index("s") * _NC + lax.axis_index("c")
    tbase = (wid % 16) * _APW
    nch = _APW // _DCH
    pltpu.sync_copy(pos_hbm.at[wid], idx_v)
    for ch in range(nch):
        par = ch % 2
        if ch >= 2:
            pltpu.make_async_copy(
                rows_v.at[par], xs_hbm.at[idx_v.at[ch - 2]], sems.at[par]
            ).wait()
        pltpu.sync_copy(x_hbm.at[pl.ds(tbase + ch * _DCH, _DCH)],
                        rows_v.at[par])
        pltpu.async_copy(rows_v.at[par], xs_hbm.at[idx_v.at[ch]],
                         sems.at[par])
    for ch in range(nch - 2, nch):
        par = ch % 2
        pltpu.make_async_copy(
            rows_v.at[par], xs_hbm.at[idx_v.at[ch]], sems.at[par]).wait()


def _w_copies(wg_hbm, wu_hbm, wd_hbm, wg_v, wu_v, wd_v, sems, e, sl):
    return (
        pltpu.make_async_copy(wg_hbm.at[e], wg_v.at[sl], sems.at[sl, 0]),
        pltpu.make_async_copy(wu_hbm.at[e], wu_v.at[sl], sems.at[sl, 1]),
        pltpu.make_async_copy(wd_hbm.at[e], wd_v.at[sl], sems.at[sl, 2]),
    )


def _gemm_body(be_ref, sl_ref, nx_ref, nb_ref, xs_ref, wg_hbm, wu_hbm,
               wd_hbm, out_ref, wg_v, wu_v, wd_v, sems):
    g = pl.program_id(0)
    cur = be_ref[g]
    prev = be_ref[jnp.maximum(g - 1, 0)]
    nxe = nx_ref[g]
    sl = sl_ref[g]
    changed = jnp.logical_or(g == 0, cur != prev)

    @pl.when(g == 0)
    def _():
        for c in _w_copies(wg_hbm, wu_hbm, wd_hbm, wg_v, wu_v, wd_v,
                           sems, cur, sl):
            c.start()

    cps = _w_copies(wg_hbm, wu_hbm, wd_hbm, wg_v, wu_v, wd_v, sems, cur, sl)

    @pl.when(jnp.logical_and(changed, nxe < E))
    def _():
        for c in _w_copies(wg_hbm, wu_hbm, wd_hbm, wg_v, wu_v, wd_v,
                           sems, nxe, 1 - sl):
            c.start()

    @pl.when(g < nb_ref[0])
    def _():
        xb = xs_ref[...]

        @pl.when(changed)
        def _():
            cps[0].wait()

        h1 = lax.dot_general(xb, wg_v[sl], (((1,), (1,)), ((), ())),
                             preferred_element_type=jnp.float32)

        @pl.when(changed)
        def _():
            cps[1].wait()

        h2 = lax.dot_general(xb, wu_v[sl], (((1,), (1,)), ((), ())),
                             preferred_element_type=jnp.float32)
        h = h1 * jax.nn.sigmoid(h1) * h2

        @pl.when(changed)
        def _():
            cps[2].wait()

        out_ref[...] = lax.dot_general(h, wd_v[sl], (((1,), (1,)), ((), ())),
                                       preferred_element_type=jnp.float32)


_gemm_call = pl.pallas_call(
    _gemm_body,
    grid_spec=pltpu.PrefetchScalarGridSpec(
        num_scalar_prefetch=4,
        grid=(G,),
        in_specs=[
            pl.BlockSpec((BLK, D), lambda g, be, sl, nx, nb: (g, 0)),
            pl.BlockSpec(memory_space=pl.ANY),
            pl.BlockSpec(memory_space=pl.ANY),
            pl.BlockSpec(memory_space=pl.ANY),
        ],
        out_specs=pl.BlockSpec((BLK, D), lambda g, be, sl, nx, nb: (g, 0)),
        scratch_shapes=[
            pltpu.VMEM((2, FF, D), jnp.float32),
            pltpu.VMEM((2, FF, D), jnp.float32),
            pltpu.VMEM((2, D, FF), jnp.float32),
            pltpu.SemaphoreType.DMA((2, 3)),
        ],
    ),
    out_shape=jax.ShapeDtypeStruct((GROWS, D), jnp.float32),
    compiler_params=pltpu.CompilerParams(
        dimension_semantics=("arbitrary",)),
)


@functools.cache
def _combine_call():
    return functools.partial(
        pl.kernel,
        mesh=_sc_mesh(),
        out_type=jax.ShapeDtypeStruct((N, D), jnp.float32),
        scratch_types=[
            pltpu.VMEM(((N // _NW) // _CCH, _CCH), jnp.int32),
            pltpu.VMEM(((N // _NW) // _CCH, _CCH), jnp.int32),
            pltpu.VMEM((2, _CCH + 16), jnp.float32),
            pltpu.VMEM((2, _CCH + 16), jnp.float32),
            pltpu.VMEM((2, _CCH, D), jnp.float32),
            pltpu.VMEM((2, _CCH, D), jnp.float32),
            pltpu.VMEM((2, _CCH, D), jnp.float32),
            pltpu.SemaphoreType.DMA((2,)),
            pltpu.SemaphoreType.DMA((2,)),
            pltpu.SemaphoreType.DMA((2,)),
        ],
    )(_combine_body)


def _combine_body(ys_hbm, p0_hbm, p1_hbm, pf_hbm, out_hbm,
                  i0_v, i1_v, q0_v, q1_v, y0_v, y1_v, ob_v, sg0, sg1, sw):
    wid = lax.axis_index("s") * _NC + lax.axis_index("c")
    tbase = wid * (N // _NW)
    nch = (N // _NW) // _CCH
    pltpu.sync_copy(p0_hbm.at[wid], i0_v)
    pltpu.sync_copy(p1_hbm.at[wid], i1_v)

    def _fetch(ch, par):
        base = tbase + ch * _CCH
        pltpu.sync_copy(pf_hbm.at[pl.ds(base, _CCH)],
                        q0_v.at[par, pl.ds(0, _CCH)])
        pltpu.sync_copy(pf_hbm.at[pl.ds(N + base, _CCH)],
                        q1_v.at[par, pl.ds(0, _CCH)])
        pltpu.async_copy(ys_hbm.at[i0_v.at[ch]], y0_v.at[par], sg0.at[par])
        pltpu.async_copy(ys_hbm.at[i1_v.at[ch]], y1_v.at[par], sg1.at[par])

    _fetch(0, 0)
    for ch in range(nch):
        par = ch % 2
        base = tbase + ch * _CCH
        if ch + 1 < nch:
            _fetch(ch + 1, 1 - par)
        pltpu.make_async_copy(ys_hbm.at[i0_v.at[ch]], y0_v.at[par],
                              sg0.at[par]).wait()
        pltpu.make_async_copy(ys_hbm.at[i1_v.at[ch]], y1_v.at[par],
                              sg1.at[par]).wait()
        if ch >= 2:
            pltpu.make_async_copy(ob_v.at[par],
                                  out_hbm.at[pl.ds(base - 2 * _CCH, _CCH)],
                                  sw.at[par]).wait()

        def _row(r, carry):
            a = q0_v[par, pl.ds(r, 16)][0]
            b = q1_v[par, pl.ds(r, 16)][0]
            for c in range(D // 16):
                ob_v[par, r, pl.ds(c * 16, 16)] = (
                    a * y0_v[par, r, pl.ds(c * 16, 16)]
                    + b * y1_v[par, r, pl.ds(c * 16, 16)])
            return carry

        lax.fori_loop(0, _CCH, _row, 0, unroll=3)
        pltpu.async_copy(ob_v.at[par], out_hbm.at[pl.ds(base, _CCH)],
                         sw.at[par])
    for ch in range(max(nch - 2, 0), nch):
        par = ch % 2
        base = tbase + ch * _CCH
        pltpu.make_async_copy(ob_v.at[par], out_hbm.at[pl.ds(base, _CCH)],
                              sw.at[par]).wait()


def kernel(x, Wg, Wgate, Wup, Wdown):
    x2d = x.reshape(N, D)
    pos, pflat, be, slot, nxte, nblk = _router_call(x2d, Wg)
    pos = pos.reshape(NA)
    pflat = pflat.reshape(NA)
    bev = be.reshape(GPAD)[:G]
    slv = slot.reshape(GPAD)[:G]
    nxv = nxte.reshape(GPAD)[:G]
    nbv = nblk.reshape(GPAD)[:1]
    pos_d = pos.reshape(_NW, _APW // _DCH, _DCH)
    nchc = (N // _NW) // _CCH
    pos0_c = pos[:N].reshape(_NW, nchc, _CCH)
    pos1_c = pos[N:].reshape(_NW, nchc, _CCH)
    xs = _dispatch_call()(x2d, pos_d)
    ys = _gemm_call(bev, slv, nxv, nbv, xs, Wgate, Wup, Wdown)
    out = _combine_call()(ys, pos0_c, pos1_c, pflat)
    return out.reshape(x.shape)


# speedup vs baseline: 1.0464x; 1.0464x over previous
import functools

import jax
import jax.numpy as jnp
from jax import lax
from jax.experimental import pallas as pl
from jax.experimental.pallas import tpu as pltpu
from jax.experimental.pallas import tpu_sc as plsc

N = 4096
D = 1024
E = 8
FF = 2048
K = 2
NA = N * K
BLK = 256
G = (NA + E * (BLK - 1) + BLK - 1) // BLK
GPAD = 128
GROWS = G * BLK

_NEG = -1e30


def _router_math(x2d, wg):
    scores = lax.dot_general(x2d, wg, (((1,), (1,)), ((), ())),
                             preferred_element_type=jnp.float32)
    iota_e = lax.broadcasted_iota(jnp.int32, (N, E), 1)
    m0 = jnp.max(scores, axis=1, keepdims=True)
    e0 = jnp.min(jnp.where(scores >= m0, iota_e, E), axis=1, keepdims=True)
    sc1 = jnp.where(iota_e == e0, _NEG, scores)
    m1 = jnp.max(sc1, axis=1, keepdims=True)
    e1 = jnp.min(jnp.where(sc1 >= m1, iota_e, E), axis=1, keepdims=True)
    z = jnp.exp(m1 - m0)
    p0 = 1.0 / (1.0 + z)
    p1 = z / (1.0 + z)

    oh0 = (iota_e == e0).astype(jnp.float32)
    oh1 = (iota_e == e1).astype(jnp.float32)
    a = jnp.concatenate([oh0, oh1], axis=0)
    incl = a
    s = 1
    while s < NA:
        incl = incl + jnp.concatenate(
            [jnp.zeros((s, E), jnp.float32), incl[:NA - s]], axis=0)
        s *= 2
    rank = jnp.sum(a * incl, axis=1, keepdims=True) - 1.0
    counts = incl[NA - 1:NA, :]
    ci = counts.astype(jnp.int32)
    pc = ((ci + (BLK - 1)) // BLK) * BLK
    pcf = pc.astype(jnp.float32)
    ex = jnp.concatenate([jnp.zeros((1, 1), jnp.float32), pcf[:, :E - 1]],
                         axis=1)
    s = 1
    while s < E:
        ex = ex + jnp.concatenate(
            [jnp.zeros((1, s), jnp.float32), ex[:, :E - s]], axis=1)
        s *= 2
    pos_f = jnp.sum(a * ex, axis=1, keepdims=True) + rank
    pos = pos_f.astype(jnp.int32)
    gstart = lax.broadcasted_iota(jnp.int32, (1, GPAD), 1).astype(
        jnp.float32) * float(BLK)
    be = jnp.zeros((1, GPAD), jnp.float32)
    for e in range(E):
        be = be + (gstart >= ex[:, e:e + 1]).astype(jnp.float32)
    be_i = be.astype(jnp.int32) - 1
    nonempty = pc > 0
    lne = jnp.zeros((1, 1), jnp.int32)
    for e in range(E):
        lne = jnp.where(nonempty[:, e:e + 1], e, lne)
    be_i = jnp.minimum(be_i, lne)
    nne_rows = []
    for e in range(E):
        nxt_e = jnp.full((1, 1), E, jnp.int32)
        for ep in range(E - 1, e, -1):
            nxt_e = jnp.where(nonempty[:, ep:ep + 1], ep, nxt_e)
        nne_rows.append(nxt_e)
    nne = jnp.concatenate(nne_rows, axis=0)
    cmp = lax.broadcasted_iota(jnp.int32, (E, GPAD), 0) == be_i
    nxte = jnp.sum(jnp.where(cmp, nne, 0), axis=0, keepdims=True)
    chg = jnp.concatenate(
        [jnp.ones((1, 1), jnp.float32),
         (be_i[:, 1:] != be_i[:, :-1]).astype(jnp.float32)], axis=1)
    csum = chg
    s = 1
    while s < GPAD:
        csum = csum + jnp.concatenate(
            [jnp.zeros((1, s), jnp.float32), csum[:, :GPAD - s]], axis=1)
        s *= 2
    slot = (csum.astype(jnp.int32) - 1) % 2
    nblk = jnp.sum(pc, axis=1, keepdims=True) // BLK
    nblk = jnp.broadcast_to(nblk, (1, GPAD))
    pflat = jnp.concatenate([p0, p1], axis=0)
    return pos, pflat, be_i, slot, nxte, nblk


def _router_body(x_ref, wg_ref, pos_ref, pf_ref, be_ref, sl_ref, nx_ref,
                 nb_ref):
    pos, pflat, be, slot, nxte, nblk = _router_math(x_ref[...], wg_ref[...])
    pos_ref[...] = pos.reshape(NA // 128, 128)
    pf_ref[...] = pflat.reshape(NA // 128, 128)
    be_ref[...] = be
    sl_ref[...] = slot
    nx_ref[...] = nxte
    nb_ref[...] = nblk


_router_call = pl.pallas_call(
    _router_body,
    out_shape=[
        jax.ShapeDtypeStruct((NA // 128, 128), jnp.int32),
        jax.ShapeDtypeStruct((NA // 128, 128), jnp.float32),
        jax.ShapeDtypeStruct((1, GPAD), jnp.int32),
        jax.ShapeDtypeStruct((1, GPAD), jnp.int32),
        jax.ShapeDtypeStruct((1, GPAD), jnp.int32),
        jax.ShapeDtypeStruct((1, GPAD), jnp.int32),
    ],
)


_NC = 2
_NS = 16
_NW = _NC * _NS
_APW = NA // _NW
_DCH = 32
_CCH = 16

@functools.cache
def _sc_mesh():
    return plsc.VectorSubcoreMesh(core_axis_name="c", subcore_axis_name="s",
                                  num_cores=_NC, num_subcores=_NS)


@functools.cache
def _dispatch_call():
    return functools.partial(
        pl.kernel,
        mesh=_sc_mesh(),
        out_type=jax.ShapeDtypeStruct((GROWS, D), jnp.float32),
        scratch_types=[
            pltpu.VMEM((_APW // _DCH, _DCH), jnp.int32),
            pltpu.VMEM((2, _DCH, D), jnp.float32),
            pltpu.SemaphoreType.DMA((2,)),
        ],
    )(_dispatch_body)


def _dispatch_body(x_hbm, pos_hbm, xs_hbm, idx_v, rows_v, sems):
    wid = lax.axis_index("s") * _NC + lax.axis_index("c")
    tbase = (wid % 16) * _APW
    nch = _APW // _DCH
    pltpu.sync_copy(pos_hbm.at[wid], idx_v)
    for ch in range(nch):
        par = ch % 2
        if ch >= 2:
            pltpu.make_async_copy(
                rows_v.at[par], xs_hbm.at[idx_v.at[ch - 2]], sems.at[par]
            ).wait()
        pltpu.sync_copy(x_hbm.at[pl.ds(tbase + ch * _DCH, _DCH)],
                        rows_v.at[par])
        pltpu.async_copy(rows_v.at[par], xs_hbm.at[idx_v.at[ch]],
                         sems.at[par])
    for ch in range(nch - 2, nch):
        par = ch % 2
        pltpu.make_async_copy(
            rows_v.at[par], xs_hbm.at[idx_v.at[ch]], sems.at[par]).wait()


def _w_copies(wg_hbm, wu_hbm, wd_hbm, wg_v, wu_v, wd_v, sems, e, sl):
    return (
        pltpu.make_async_copy(wg_hbm.at[e], wg_v.at[sl], sems.at[sl, 0]),
        pltpu.make_async_copy(wu_hbm.at[e], wu_v.at[sl], sems.at[sl, 1]),
        pltpu.make_async_copy(wd_hbm.at[e], wd_v.at[sl], sems.at[sl, 2]),
    )


def _gemm_body(be_ref, sl_ref, nx_ref, nb_ref, xs_ref, wg_hbm, wu_hbm,
               wd_hbm, out_ref, wg_v, wu_v, wd_v, sems):
    g = pl.program_id(0)
    cur = be_ref[g]
    prev = be_ref[jnp.maximum(g - 1, 0)]
    nxe = nx_ref[g]
    sl = sl_ref[g]
    changed = jnp.logical_or(g == 0, cur != prev)

    @pl.when(g == 0)
    def _():
        for c in _w_copies(wg_hbm, wu_hbm, wd_hbm, wg_v, wu_v, wd_v,
                           sems, cur, sl):
            c.start()

    @pl.when(changed)
    def _():
        for c in _w_copies(wg_hbm, wu_hbm, wd_hbm, wg_v, wu_v, wd_v,
                           sems, cur, sl):
            c.wait()

    @pl.when(jnp.logical_and(changed, nxe < E))
    def _():
        for c in _w_copies(wg_hbm, wu_hbm, wd_hbm, wg_v, wu_v, wd_v,
                           sems, nxe, 1 - sl):
            c.start()

    @pl.when(g < nb_ref[0])
    def _():
        xb = xs_ref[...]
        h1 = lax.dot_general(xb, wg_v[sl], (((1,), (1,)), ((), ())),
                             preferred_element_type=jnp.float32)
        h2 = lax.dot_general(xb, wu_v[sl], (((1,), (1,)), ((), ())),
                             preferred_element_type=jnp.float32)
        h = h1 * jax.nn.sigmoid(h1) * h2
        out_ref[...] = lax.dot_general(h, wd_v[sl], (((1,), (1,)), ((), ())),
                                       preferred_element_type=jnp.float32)


_gemm_call = pl.pallas_call(
    _gemm_body,
    grid_spec=pltpu.PrefetchScalarGridSpec(
        num_scalar_prefetch=4,
        grid=(G,),
        in_specs=[
            pl.BlockSpec((BLK, D), lambda g, be, sl, nx, nb: (g, 0)),
            pl.BlockSpec(memory_space=pl.ANY),
            pl.BlockSpec(memory_space=pl.ANY),
            pl.BlockSpec(memory_space=pl.ANY),
        ],
        out_specs=pl.BlockSpec((BLK, D), lambda g, be, sl, nx, nb: (g, 0)),
        scratch_shapes=[
            pltpu.VMEM((2, FF, D), jnp.float32),
            pltpu.VMEM((2, FF, D), jnp.float32),
            pltpu.VMEM((2, D, FF), jnp.float32),
            pltpu.SemaphoreType.DMA((2, 3)),
        ],
    ),
    out_shape=jax.ShapeDtypeStruct((GROWS, D), jnp.float32),
    compiler_params=pltpu.CompilerParams(
        dimension_semantics=("arbitrary",)),
)


@functools.cache
def _combine_call():
    return functools.partial(
        pl.kernel,
        mesh=_sc_mesh(),
        out_type=jax.ShapeDtypeStruct((N, D), jnp.float32),
        scratch_types=[
            pltpu.VMEM(((N // _NW) // _CCH, _CCH), jnp.int32),
            pltpu.VMEM(((N // _NW) // _CCH, _CCH), jnp.int32),
            pltpu.VMEM((2, _CCH + 16), jnp.float32),
            pltpu.VMEM((2, _CCH + 16), jnp.float32),
            pltpu.VMEM((2, _CCH, D), jnp.float32),
            pltpu.VMEM((2, _CCH, D), jnp.float32),
            pltpu.VMEM((2, _CCH, D), jnp.float32),
            pltpu.SemaphoreType.DMA((2,)),
            pltpu.SemaphoreType.DMA((2,)),
            pltpu.SemaphoreType.DMA((2,)),
        ],
    )(_combine_body)


def _combine_body(ys_hbm, p0_hbm, p1_hbm, pf_hbm, out_hbm,
                  i0_v, i1_v, q0_v, q1_v, y0_v, y1_v, ob_v, sg0, sg1, sw):
    wid = lax.axis_index("s") * _NC + lax.axis_index("c")
    tbase = wid * (N // _NW)
    nch = (N // _NW) // _CCH
    pltpu.sync_copy(p0_hbm.at[wid], i0_v)
    pltpu.sync_copy(p1_hbm.at[wid], i1_v)

    def _fetch(ch, par):
        base = tbase + ch * _CCH
        pltpu.sync_copy(pf_hbm.at[pl.ds(base, _CCH)],
                        q0_v.at[par, pl.ds(0, _CCH)])
        pltpu.sync_copy(pf_hbm.at[pl.ds(N + base, _CCH)],
                        q1_v.at[par, pl.ds(0, _CCH)])
        pltpu.async_copy(ys_hbm.at[i0_v.at[ch]], y0_v.at[par], sg0.at[par])
        pltpu.async_copy(ys_hbm.at[i1_v.at[ch]], y1_v.at[par], sg1.at[par])

    _fetch(0, 0)
    for ch in range(nch):
        par = ch % 2
        base = tbase + ch * _CCH
        if ch + 1 < nch:
            _fetch(ch + 1, 1 - par)
        pltpu.make_async_copy(ys_hbm.at[i0_v.at[ch]], y0_v.at[par],
                              sg0.at[par]).wait()
        pltpu.make_async_copy(ys_hbm.at[i1_v.at[ch]], y1_v.at[par],
                              sg1.at[par]).wait()
        if ch >= 2:
            pltpu.make_async_copy(ob_v.at[par],
                                  out_hbm.at[pl.ds(base - 2 * _CCH, _CCH)],
                                  sw.at[par]).wait()

        def _row(r, carry):
            a = q0_v[par, pl.ds(r, 16)][0]
            b = q1_v[par, pl.ds(r, 16)][0]
            for c in range(D // 16):
                ob_v[par, r, pl.ds(c * 16, 16)] = (
                    a * y0_v[par, r, pl.ds(c * 16, 16)]
                    + b * y1_v[par, r, pl.ds(c * 16, 16)])
            return carry

        lax.fori_loop(0, _CCH, _row, 0, unroll=3)
        pltpu.async_copy(ob_v.at[par], out_hbm.at[pl.ds(base, _CCH)],
                         sw.at[par])
    for ch in range(max(nch - 2, 0), nch):
        par = ch % 2
        base = tbase + ch * _CCH
        pltpu.make_async_copy(ob_v.at[par], out_hbm.at[pl.ds(base, _CCH)],
                              sw.at[par]).wait()


def kernel(x, Wg, Wgate, Wup, Wdown):
    x2d = x.reshape(N, D)
    pos, pflat, be, slot, nxte, nblk = _router_call(x2d, Wg)
    pos = pos.reshape(NA)
    pflat = pflat.reshape(NA)
    bev = be.reshape(GPAD)[:G]
    slv = slot.reshape(GPAD)[:G]
    nxv = nxte.reshape(GPAD)[:G]
    nbv = nblk.reshape(GPAD)[:1]
    pos_d = pos.reshape(_NW, _APW // _DCH, _DCH)
    nchc = (N // _NW) // _CCH
    pos0_c = pos[:N].reshape(_NW, nchc, _CCH)
    pos1_c = pos[N:].reshape(_NW, nchc, _CCH)
    xs = _dispatch_call()(x2d, pos_d)
    ys = _gemm_call(bev, slv, nxv, nbv, xs, Wgate, Wup, Wdown)
    out = _combine_call()(ys, pos0_c, pos1_c, pflat)
    return out.reshape(x.shape)
